# baseline (speedup 1.0000x reference)
import jax
import jax.numpy as jnp
from jax.experimental import pallas as pl
from jax.experimental.pallas import tpu as pltpu

_HIDDEN = 768
_RANK = _HIDDEN // 4
_THRESH = 0.5
_N = 4 * 8192
_B = 1024
_NB = _N // _B


def _gate_kernel(x_ref, wt_ref, mask_ref, score_ref):
    x = x_ref[...]
    h = jnp.dot(x, wt_ref[...], preferred_element_type=jnp.float32)
    s2 = jnp.sum(h * h, axis=-1, keepdims=True)
    score = jnp.sqrt(s2)
    m = score >= _THRESH
    mask_ref[...] = m.astype(jnp.float32)
    score_ref[...] = jnp.where(m, score, -jnp.inf)


def kernel(x, W):
    xf = x.reshape(_N, _HIDDEN)
    wt = W.T
    mask_f, score = pl.pallas_call(
        _gate_kernel,
        grid=(_NB,),
        in_specs=[
            pl.BlockSpec((_B, _HIDDEN), lambda i: (i, 0)),
            pl.BlockSpec((_HIDDEN, _RANK), lambda i: (0, 0)),
        ],
        out_specs=[
            pl.BlockSpec((_B, 1), lambda i: (i, 0)),
            pl.BlockSpec((_B, 1), lambda i: (i, 0)),
        ],
        out_shape=[
            jax.ShapeDtypeStruct((_N, 1), jnp.float32),
            jax.ShapeDtypeStruct((_N, 1), jnp.float32),
        ],
        compiler_params=pltpu.CompilerParams(
            dimension_semantics=("parallel",),
        ),
    )(xf, wt)
    lead = x.shape[:-1]
    return mask_f.reshape(lead).astype(jnp.bool_), score.reshape(lead)


# speedup vs baseline: 1.0720x; 1.0720x over previous
import jax
import jax.numpy as jnp
from jax.experimental import pallas as pl
from jax.experimental.pallas import tpu as pltpu

_HIDDEN = 768
_RANK = _HIDDEN // 4
_THRESH = 0.5
_N = 4 * 8192
_CH = 512
_NCH = _N // _CH
_NBUF = 12


def _start_copy(x_hbm, xbuf, sems, chunk, slot):
    pltpu.make_async_copy(
        x_hbm.at[pl.ds(chunk * _CH, _CH), :],
        xbuf.at[slot],
        sems.at[slot],
    ).start()


def _gate_kernel(x_hbm, wt_ref, mask_ref, score_ref, xbuf, sems):
    i = pl.program_id(0)

    @pl.when(i == 0)
    def _():
        for b in range(_NBUF - 1):
            _start_copy(x_hbm, xbuf, sems, b, b)

    nxt = i + _NBUF - 1

    @pl.when(nxt < _NCH)
    def _():
        _start_copy(x_hbm, xbuf, sems, nxt, jax.lax.rem(nxt, _NBUF))

    slot = jax.lax.rem(i, _NBUF)
    pltpu.make_async_copy(
        x_hbm.at[pl.ds(i * _CH, _CH), :],
        xbuf.at[slot],
        sems.at[slot],
    ).wait()

    x = xbuf[slot]
    h = jnp.dot(x, wt_ref[...], preferred_element_type=jnp.float32)
    s2 = jnp.sum(h * h, axis=-1, keepdims=True)
    score = jnp.sqrt(s2)
    m = score >= _THRESH
    mask_ref[...] = m.astype(jnp.float32)
    score_ref[...] = jnp.where(m, score, -jnp.inf)


def kernel(x, W):
    xf = x.reshape(_N, _HIDDEN)
    wt = W.T
    mask_f, score = pl.pallas_call(
        _gate_kernel,
        grid=(_NCH,),
        in_specs=[
            pl.BlockSpec(memory_space=pl.ANY),
            pl.BlockSpec((_HIDDEN, _RANK), lambda i: (0, 0)),
        ],
        out_specs=[
            pl.BlockSpec((_CH, 1), lambda i: (i, 0)),
            pl.BlockSpec((_CH, 1), lambda i: (i, 0)),
        ],
        out_shape=[
            jax.ShapeDtypeStruct((_N, 1), jnp.float32),
            jax.ShapeDtypeStruct((_N, 1), jnp.float32),
        ],
        scratch_shapes=[
            pltpu.VMEM((_NBUF, _CH, _HIDDEN), jnp.float32),
            pltpu.SemaphoreType.DMA((_NBUF,)),
        ],
    )(xf, wt)
    lead = x.shape[:-1]
    return mask_f.reshape(lead).astype(jnp.bool_), score.reshape(lead)


# speedup vs baseline: 1.0787x; 1.0063x over previous
import jax
import jax.numpy as jnp
from jax.experimental import pallas as pl
from jax.experimental.pallas import tpu as pltpu

_HIDDEN = 768
_RANK = _HIDDEN // 4
_THRESH = 0.5
_N = 4 * 8192
_CH = 512
_NCH = _N // _CH
_NBUF = 12


def _start_copy(x_hbm, xbuf, sems, chunk, slot):
    pltpu.make_async_copy(
        x_hbm.at[pl.ds(chunk * _CH, _CH), :],
        xbuf.at[slot],
        sems.at[slot],
    ).start()


def _gate_kernel(x_hbm, wt_ref, mask_ref, score_ref, xbuf, sems):
    i = pl.program_id(0)

    @pl.when(i == 0)
    def _():
        for b in range(_NBUF - 1):
            _start_copy(x_hbm, xbuf, sems, b, b)

    nxt = i + _NBUF - 1

    @pl.when(nxt < _NCH)
    def _():
        _start_copy(x_hbm, xbuf, sems, nxt, jax.lax.rem(nxt, _NBUF))

    slot = jax.lax.rem(i, _NBUF)
    pltpu.make_async_copy(
        x_hbm.at[pl.ds(i * _CH, _CH), :],
        xbuf.at[slot],
        sems.at[slot],
    ).wait()

    x = xbuf[slot].astype(jnp.bfloat16)
    h = jnp.dot(x, wt_ref[...], preferred_element_type=jnp.float32)
    s2 = jnp.sum(h * h, axis=-1, keepdims=True)
    score = jnp.sqrt(s2)
    m = score >= _THRESH
    mask_ref[...] = m.astype(jnp.float32)
    score_ref[...] = jnp.where(m, score, -jnp.inf)


def kernel(x, W):
    xf = x.reshape(_N, _HIDDEN)
    wt = W.T.astype(jnp.bfloat16)
    mask_f, score = pl.pallas_call(
        _gate_kernel,
        grid=(_NCH,),
        in_specs=[
            pl.BlockSpec(memory_space=pl.ANY),
            pl.BlockSpec((_HIDDEN, _RANK), lambda i: (0, 0)),
        ],
        out_specs=[
            pl.BlockSpec((_CH, 1), lambda i: (i, 0)),
            pl.BlockSpec((_CH, 1), lambda i: (i, 0)),
        ],
        out_shape=[
            jax.ShapeDtypeStruct((_N, 1), jnp.float32),
            jax.ShapeDtypeStruct((_N, 1), jnp.float32),
        ],
        scratch_shapes=[
            pltpu.VMEM((_NBUF, _CH, _HIDDEN), jnp.float32),
            pltpu.SemaphoreType.DMA((_NBUF,)),
        ],
    )(xf, wt)
    lead = x.shape[:-1]
    return mask_f.reshape(lead).astype(jnp.bool_), score.reshape(lead)


# speedup vs baseline: 1.1532x; 1.0691x over previous
import jax
import jax.numpy as jnp
from jax.experimental import pallas as pl
from jax.experimental.pallas import tpu as pltpu

_HIDDEN = 768
_RANK = _HIDDEN // 4
_THRESH = 0.5
_N = 4 * 8192
_CH = 2048
_NCH = _N // _CH
_NBUF = 4


def _start_copy(x_hbm, xbuf, sems, chunk, slot):
    pltpu.make_async_copy(
        x_hbm.at[pl.ds(chunk * _CH, _CH), :],
        xbuf.at[slot],
        sems.at[slot],
    ).start()


def _gate_kernel(x_hbm, wt_ref, mask_ref, score_ref, xbuf, sems):
    i = pl.program_id(0)

    @pl.when(i == 0)
    def _():
        for b in range(_NBUF - 1):
            _start_copy(x_hbm, xbuf, sems, b, b)

    nxt = i + _NBUF - 1

    @pl.when(nxt < _NCH)
    def _():
        _start_copy(x_hbm, xbuf, sems, nxt, jax.lax.rem(nxt, _NBUF))

    slot = jax.lax.rem(i, _NBUF)
    pltpu.make_async_copy(
        x_hbm.at[pl.ds(i * _CH, _CH), :],
        xbuf.at[slot],
        sems.at[slot],
    ).wait()

    x = xbuf[slot]
    h = jnp.dot(x, wt_ref[...], preferred_element_type=jnp.float32)
    s2 = jnp.sum(h * h, axis=-1, keepdims=True)
    score = jnp.sqrt(s2)
    m = score >= _THRESH
    mask_ref[...] = m.astype(jnp.float32)
    score_ref[...] = jnp.where(m, score, -jnp.inf)


def kernel(x, W):
    xf = x.reshape(_N, _HIDDEN)
    wt = W.T
    mask_f, score = pl.pallas_call(
        _gate_kernel,
        grid=(_NCH,),
        in_specs=[
            pl.BlockSpec(memory_space=pl.ANY),
            pl.BlockSpec((_HIDDEN, _RANK), lambda i: (0, 0)),
        ],
        out_specs=[
            pl.BlockSpec((_CH, 1), lambda i: (i, 0)),
            pl.BlockSpec((_CH, 1), lambda i: (i, 0)),
        ],
        out_shape=[
            jax.ShapeDtypeStruct((_N, 1), jnp.float32),
            jax.ShapeDtypeStruct((_N, 1), jnp.float32),
        ],
        scratch_shapes=[
            pltpu.VMEM((_NBUF, _CH, _HIDDEN), jnp.float32),
            pltpu.SemaphoreType.DMA((_NBUF,)),
        ],
    )(xf, wt)
    lead = x.shape[:-1]
    return mask_f.reshape(lead).astype(jnp.bool_), score.reshape(lead)
